# baseline (speedup 1.0000x reference)
import functools

import numpy as np
import jax
import jax.numpy as jnp
from jax import lax
from jax.experimental import pallas as pl
from jax.experimental.pallas import tpu as pltpu
from jax.experimental.pallas import tpu_sc as plsc

_B, _N, _D, _H, _W, _C = 2, 6, 59, 16, 44, 80
_NP = _B * _N * _D * _H * _W
_NPB = _NP // _B
_XG, _YG, _ZG = 128, 128, 1
_ROWS = _XG * _YG
_PAD_ROWS = 16
_ACC_ROWS = _ROWS + _PAD_ROWS
_BLK = 176
_NG = _BLK // 16
_BPI = _H * _W // _BLK
_NBLK = _NPB // _BLK
_NH = _N // 2
_NBLKH = _NH * _D * _BPI
_NTILES = 16
_MPTH = 45
_OUTER = 16
_WB_ROWS = _ROWS // _NTILES
_NSLOT = 3

_DX = np.array([0.8, 0.8, 8.0], dtype=np.float32)
_BX = np.array([-51.2 + 0.4, -51.2 + 0.4, -5.0 + 4.0], dtype=np.float32)
_C0 = _BX - _DX / np.float32(2.0)

_mesh = plsc.VectorSubcoreMesh(core_axis_name="c", subcore_axis_name="s")

_scratch = (
    [pltpu.VMEM((_NSLOT, 3, _BLK), jnp.float32)]
    + [pltpu.VMEM((_NSLOT, _BLK, _C), jnp.float32)]
    + [pltpu.VMEM((_NSLOT, _BLK), jnp.int32)]
    + [pltpu.VMEM_SHARED((_ACC_ROWS, _C), jnp.float32)]
    + [pltpu.SemaphoreType.DMA] * (3 * _NSLOT)
)


def _make_half(blk_off, init_partial):
    @functools.partial(
        pl.kernel,
        mesh=_mesh,
        out_type=jax.ShapeDtypeStruct((_B, _ROWS, _C), jnp.float32),
        scratch_types=_scratch,
        compiler_params=pltpu.CompilerParams(use_tc_tiling_on_sc=False),
    )
    def _half(gT_hbm, xP_hbm, init_hbm, out_hbm, gbuf, fbuf, ibuf, acc, *sems):
        gsem = sems[0:_NSLOT]
        fsem = sems[_NSLOT:2 * _NSLOT]
        ssem = sems[2 * _NSLOT:3 * _NSLOT]
        c = lax.axis_index("c")
        s = lax.axis_index("s")

        wb = s * _WB_ROWS
        if init_partial:
            pltpu.sync_copy(init_hbm.at[c, pl.ds(wb, _WB_ROWS)],
                            acc.at[pl.ds(wb, _WB_ROWS)])
        else:
            for r in range(_WB_ROWS // 128):
                pltpu.sync_copy(init_hbm, acc.at[pl.ds(wb + r * 128, 128)])
        plsc.subcore_barrier()

        def _gathers(m, b):
            lb = s + _NTILES * m
            gb = c * _NBLK + blk_off + lb
            img = lb >> 2
            n = (img * 1111) >> 16
            d = img - n * 59
            p0 = (lb & (_BPI - 1)) * _BLK
            return [
                pltpu.make_async_copy(
                    gT_hbm.at[:, pl.ds(gb * _BLK, _BLK)], gbuf.at[b], gsem[b]),
                pltpu.make_async_copy(
                    xP_hbm.at[c, n, d, pl.ds(p0, _BLK)], fbuf.at[b], fsem[b]),
            ]

        def _scatter(b):
            return pltpu.make_async_copy(
                fbuf.at[b], acc.at[ibuf.at[b]], ssem[b])

        for b in range(_NSLOT - 1):
            for d in _gathers(b, b):
                d.start()

        def outer(i, carry):
            for b in range(_NSLOT):
                m = i * _NSLOT + b
                lb = s + _NTILES * m

                @pl.when(lb < _NBLKH)
                def _(b=b, m=m):
                    for d in _gathers(m, b):
                        d.wait()
                    for j in range(_NG):
                        sl = pl.ds(j * 16, 16)
                        ix = ((gbuf[b, 0, sl] - _C0[0]) / _DX[0]).astype(jnp.int32)
                        iy = ((gbuf[b, 1, sl] - _C0[1]) / _DX[1]).astype(jnp.int32)
                        iz = ((gbuf[b, 2, sl] - _C0[2]) / _DX[2]).astype(jnp.int32)
                        kept = ((ix >= 0) & (ix < _XG) & (iy >= 0) & (iy < _YG)
                                & (iz >= 0) & (iz < _ZG))
                        ibuf[b, sl] = jnp.where(kept, ix * _YG + iy, _ROWS + s)
                    _scatter(b).start(add=True)

                bn = (b + 2) % _NSLOT
                lbd = s + _NTILES * (m - 1)
                lbp = s + _NTILES * (m + 2)

                @pl.when((m >= 1) & (lbd < _NBLKH))
                def _(bn=bn):
                    _scatter(bn).wait()

                @pl.when(lbp < _NBLKH)
                def _(bn=bn, mp=m + 2):
                    for d in _gathers(mp, bn):
                        d.start()

            return carry

        lax.fori_loop(0, _OUTER, outer, 0)
        plsc.subcore_barrier()

        pltpu.sync_copy(acc.at[pl.ds(wb, _WB_ROWS)],
                        out_hbm.at[c, pl.ds(wb, _WB_ROWS)])

    return _half


_KA = _make_half(0, False)
_KB = _make_half(_NBLKH, True)


def kernel(geom_feats, x):
    B, N, D, H, W, C = x.shape
    assert (B, N, D, H, W, C) == (_B, _N, _D, _H, _W, _C)
    gT = jnp.transpose(geom_feats, (5, 0, 1, 2, 4, 3)).reshape(3, _NP)
    xdep, gT = lax.optimization_barrier((x, gT))
    xPa = jnp.transpose(xdep[:, :_NH], (0, 1, 2, 4, 3, 5)).reshape(
        _B, _NH, _D, H * W, C)
    xPb = jnp.transpose(xdep[:, _NH:], (0, 1, 2, 4, 3, 5)).reshape(
        _B, _NH, _D, H * W, C)
    zeros = jnp.zeros((128, C), jnp.float32)
    partial = _KA(gT, xPa, zeros)
    out = _KB(gT, xPb, partial)
    return out.reshape(B, _XG, _YG, C).transpose(0, 3, 1, 2)


# speedup vs baseline: 1.3315x; 1.3315x over previous
import functools

import numpy as np
import jax
import jax.numpy as jnp
from jax import lax
from jax.experimental import pallas as pl
from jax.experimental.pallas import tpu as pltpu
from jax.experimental.pallas import tpu_sc as plsc

_B, _N, _D, _H, _W, _C = 2, 6, 59, 16, 44, 80
_NP = _B * _N * _D * _H * _W
_NPB = _NP // _B
_XG, _YG, _ZG = 128, 128, 1
_ROWS = _XG * _YG
_PAD_ROWS = 16
_ACC_ROWS = _ROWS + _PAD_ROWS
_BLK = 176
_NG = _BLK // 16
_BPI = _H * _W // _BLK
_NBLK = _NPB // _BLK
_NTILES = 16
_OUTER = 30
_WB_ROWS = _ROWS // _NTILES
_ZERO_ROWS = _ACC_ROWS // _NTILES
_NSLOT = 3

_DX = np.array([0.8, 0.8, 8.0], dtype=np.float32)
_BX = np.array([-51.2 + 0.4, -51.2 + 0.4, -5.0 + 4.0], dtype=np.float32)
_C0 = _BX - _DX / np.float32(2.0)

_mesh = plsc.VectorSubcoreMesh(core_axis_name="c", subcore_axis_name="s")

_scratch = (
    [pltpu.VMEM((_NSLOT, 3, _BLK), jnp.float32)]
    + [pltpu.VMEM((_NSLOT, _BLK, _C), jnp.float32)]
    + [pltpu.VMEM((_NSLOT, _BLK), jnp.int32)]
    + [pltpu.VMEM_SHARED((_ACC_ROWS, _C), jnp.float32)]
    + [pltpu.SemaphoreType.DMA] * (3 * _NSLOT)
)


@functools.partial(
    pl.kernel,
    mesh=_mesh,
    out_type=jax.ShapeDtypeStruct((_B, _ROWS, _C), jnp.float32),
    scratch_types=_scratch,
    compiler_params=pltpu.CompilerParams(use_tc_tiling_on_sc=False),
)
def _bev_pool_sc(gT_hbm, xP_hbm, zeros_hbm, out_hbm, gbuf, fbuf, ibuf, acc, *sems):
    gsem = sems[0:_NSLOT]
    fsem = sems[_NSLOT:2 * _NSLOT]
    ssem = sems[2 * _NSLOT:3 * _NSLOT]
    c = lax.axis_index("c")
    s = lax.axis_index("s")

    zbase = s * _ZERO_ROWS
    for r in range(_ZERO_ROWS // 128):
        pltpu.sync_copy(zeros_hbm, acc.at[pl.ds(zbase + r * 128, 128)])
    rem = _ZERO_ROWS % 128
    if rem:
        pltpu.sync_copy(zeros_hbm.at[pl.ds(0, rem)],
                        acc.at[pl.ds(zbase + _ZERO_ROWS - rem, rem)])
    plsc.subcore_barrier()

    def _gathers(m, b):
        lb = s + _NTILES * m
        gb = c * _NBLK + lb
        img = lb >> 2
        n = (img * 1111) >> 16
        d = img - n * 59
        p0 = (lb & (_BPI - 1)) * _BLK
        return [
            pltpu.make_async_copy(
                gT_hbm.at[:, pl.ds(gb * _BLK, _BLK)], gbuf.at[b], gsem[b]),
            pltpu.make_async_copy(
                xP_hbm.at[c, n, d, pl.ds(p0, _BLK)], fbuf.at[b], fsem[b]),
        ]

    def _scatter(b):
        return pltpu.make_async_copy(fbuf.at[b], acc.at[ibuf.at[b]], ssem[b])

    for b in range(_NSLOT - 1):
        for d in _gathers(b, b):
            d.start()

    def outer(i, carry):
        for b in range(_NSLOT):
            m = i * _NSLOT + b
            lb = s + _NTILES * m

            @pl.when(lb < _NBLK)
            def _(b=b, m=m):
                for d in _gathers(m, b):
                    d.wait()
                for j in range(_NG):
                    sl = pl.ds(j * 16, 16)
                    ix = ((gbuf[b, 0, sl] - _C0[0]) / _DX[0]).astype(jnp.int32)
                    iy = ((gbuf[b, 1, sl] - _C0[1]) / _DX[1]).astype(jnp.int32)
                    iz = ((gbuf[b, 2, sl] - _C0[2]) / _DX[2]).astype(jnp.int32)
                    kept = ((ix >= 0) & (ix < _XG) & (iy >= 0) & (iy < _YG)
                            & (iz >= 0) & (iz < _ZG))
                    ibuf[b, sl] = jnp.where(kept, ix * _YG + iy, _ROWS + s)
                _scatter(b).start(add=True)

            bn = (b + 2) % _NSLOT
            lbd = s + _NTILES * (m - 1)
            lbp = s + _NTILES * (m + 2)

            @pl.when((m >= 1) & (lbd < _NBLK))
            def _(bn=bn):
                _scatter(bn).wait()

            @pl.when(lbp < _NBLK)
            def _(bn=bn, mp=m + 2):
                for d in _gathers(mp, bn):
                    d.start()

        return carry

    lax.fori_loop(0, _OUTER, outer, 0)
    plsc.subcore_barrier()

    wb = s * _WB_ROWS
    pltpu.sync_copy(acc.at[pl.ds(wb, _WB_ROWS)],
                    out_hbm.at[c, pl.ds(wb, _WB_ROWS)])


def kernel(geom_feats, x):
    B, N, D, H, W, C = x.shape
    assert (B, N, D, H, W, C) == (_B, _N, _D, _H, _W, _C)
    xP = jnp.transpose(x, (0, 1, 2, 4, 3, 5)).reshape(_B, _N, _D, H * W, C)
    gT = jnp.transpose(geom_feats, (5, 0, 1, 2, 4, 3)).reshape(3, _NP)
    zeros = jnp.zeros((128, C), jnp.float32)
    out = _bev_pool_sc(gT, xP, zeros)
    return out.reshape(B, _XG, _YG, C).transpose(0, 3, 1, 2)
